# Initial kernel scaffold; baseline (speedup 1.0000x reference)
import jax
import jax.numpy as jnp
from jax.experimental import pallas as pl


def kernel(vertices, edges, W, b):
    raise NotImplementedError("write your pallas kernel here")


# speedup vs baseline: 5.8071x; 5.8071x over previous
import functools

import jax
import jax.numpy as jnp
from jax import lax
from jax.experimental import pallas as pl
from jax.experimental.pallas import tpu as pltpu
from jax.experimental.pallas import tpu_sc as plsc

N_NODES = 10000
D = 128
N_EDGES = 320000

NC = 2
NS = 16
TILES = NC * NS
CHUNK = 128
CHUNKS_PER_TILE = 80
PER_TILE = CHUNK * CHUNKS_PER_TILE
TOT_EDGES = PER_TILE * TILES
NODES_PAD = 10240
DEG_W = 16
ROWS_PER_TILE = NODES_PAD // NS


def _sc_aggregate(src_hbm, dst_hbm, verts_hbm, agg_out,
                  src_v, dst_v, rows_v, agg_sh, sem):
    c = lax.axis_index("c")
    s = lax.axis_index("s")
    wid = s * NC + c

    zrow = jnp.zeros((16,), jnp.float32)
    def _zero_body(i, _):
        for k in range(D // 16):
            rows_v[i, pl.ds(16 * k, 16)] = zrow
        return 0
    lax.fori_loop(0, CHUNK, _zero_body, 0)
    for r in range(ROWS_PER_TILE // CHUNK):
        pltpu.sync_copy(rows_v, agg_sh.at[pl.ds(s * ROWS_PER_TILE + r * CHUNK, CHUNK)])

    pltpu.sync_copy(src_hbm.at[wid], src_v)
    pltpu.sync_copy(dst_hbm.at[wid], dst_v)

    plsc.subcore_barrier()

    def _chunk_body(j, _):
        pltpu.async_copy(verts_hbm.at[src_v.at[j]], rows_v, sem).wait()
        pltpu.sync_copy(rows_v, agg_sh.at[dst_v.at[j]], add=True)
        return 0
    lax.fori_loop(0, CHUNKS_PER_TILE, _chunk_body, 0)

    plsc.subcore_barrier()

    rbase = s * ROWS_PER_TILE
    pltpu.sync_copy(agg_sh.at[pl.ds(rbase, ROWS_PER_TILE)],
                    agg_out.at[c].at[pl.ds(rbase, ROWS_PER_TILE)])


def _sc_degrees(dst_hbm, ones_hbm, deg_out, dst_v, ones_v, deg_sh):
    c = lax.axis_index("c")
    s = lax.axis_index("s")
    wid = s * NC + c

    zrow = jnp.zeros((16,), jnp.float32)
    def _zero_body(i, _):
        ones_v[i, :] = zrow
        return 0
    lax.fori_loop(0, CHUNK, _zero_body, 0)
    for r in range(ROWS_PER_TILE // CHUNK):
        pltpu.sync_copy(ones_v, deg_sh.at[pl.ds(s * ROWS_PER_TILE + r * CHUNK, CHUNK)])

    pltpu.sync_copy(dst_hbm.at[wid], dst_v)
    pltpu.sync_copy(ones_hbm, ones_v)

    plsc.subcore_barrier()

    def _chunk_body(j, _):
        pltpu.sync_copy(ones_v, deg_sh.at[dst_v.at[j]], add=True)
        return 0
    lax.fori_loop(0, CHUNKS_PER_TILE, _chunk_body, 0)

    plsc.subcore_barrier()

    rbase = s * ROWS_PER_TILE
    pltpu.sync_copy(deg_sh.at[pl.ds(rbase, ROWS_PER_TILE)],
                    deg_out.at[c].at[pl.ds(rbase, ROWS_PER_TILE)])


def _tc_finish(p0, p1, v, d0, d1, wt, bb, o):
    deg = d0[:, 0:1] + d1[:, 0:1] + 1.0
    x = (p0[...] + p1[...] + v[...]) * lax.rsqrt(deg)
    o[...] = jnp.dot(x, wt[...], preferred_element_type=jnp.float32) + bb[...]


def kernel(vertices, edges, W, b):
    pad = TOT_EDGES - N_EDGES
    src = jnp.concatenate([edges[1], jnp.zeros((pad,), jnp.int32)])
    dst = jnp.concatenate([edges[0], jnp.full((pad,), N_NODES, jnp.int32)])
    src = src.reshape(TILES, CHUNKS_PER_TILE, CHUNK)
    dst = dst.reshape(TILES, CHUNKS_PER_TILE, CHUNK)
    ones = jnp.ones((CHUNK, DEG_W), jnp.float32)

    mesh = plsc.VectorSubcoreMesh(core_axis_name="c", subcore_axis_name="s")

    agg = functools.partial(
        pl.kernel,
        mesh=mesh,
        out_type=jax.ShapeDtypeStruct((NC, NODES_PAD, D), jnp.float32),
        scratch_types=[
            pltpu.VMEM((CHUNKS_PER_TILE, CHUNK), jnp.int32),
            pltpu.VMEM((CHUNKS_PER_TILE, CHUNK), jnp.int32),
            pltpu.VMEM((CHUNK, D), jnp.float32),
            pltpu.VMEM_SHARED((NODES_PAD, D), jnp.float32),
            pltpu.SemaphoreType.DMA,
        ],
    )(_sc_aggregate)(src, dst, vertices)

    deg = functools.partial(
        pl.kernel,
        mesh=mesh,
        out_type=jax.ShapeDtypeStruct((NC, NODES_PAD, DEG_W), jnp.float32),
        scratch_types=[
            pltpu.VMEM((CHUNKS_PER_TILE, CHUNK), jnp.int32),
            pltpu.VMEM((CHUNK, DEG_W), jnp.float32),
            pltpu.VMEM_SHARED((NODES_PAD, DEG_W), jnp.float32),
        ],
    )(_sc_degrees)(dst, ones)

    rows_blk = 1000
    grid = (N_NODES // rows_blk,)
    out = pl.pallas_call(
        _tc_finish,
        grid=grid,
        in_specs=[
            pl.BlockSpec((rows_blk, D), lambda i: (i, 0)),
            pl.BlockSpec((rows_blk, D), lambda i: (i, 0)),
            pl.BlockSpec((rows_blk, D), lambda i: (i, 0)),
            pl.BlockSpec((rows_blk, DEG_W), lambda i: (i, 0)),
            pl.BlockSpec((rows_blk, DEG_W), lambda i: (i, 0)),
            pl.BlockSpec((D, D), lambda i: (0, 0)),
            pl.BlockSpec((1, D), lambda i: (0, 0)),
        ],
        out_specs=pl.BlockSpec((rows_blk, D), lambda i: (i, 0)),
        out_shape=jax.ShapeDtypeStruct((N_NODES, D), jnp.float32),
    )(
        agg[0, :N_NODES], agg[1, :N_NODES], vertices,
        deg[0, :N_NODES], deg[1, :N_NODES],
        W.T, b.reshape(1, D),
    )
    return out
